# baseline (speedup 1.0000x reference)
import functools
import math

import numpy as np
import jax
import jax.numpy as jnp
from jax import lax
from jax.experimental import pallas as pl
from jax.experimental.pallas import tpu as pltpu
from jax.experimental.pallas import tpu_sc as plsc

B = 8
PER = 256
FEAT = 512
TDIM = 256
TRI = PER * (PER - 1) // 2
WORKERS_PER_GRAPH = 4
OUT_PER_W = TRI // WORKERS_PER_GRAPH
OUT_PAD = 8192
WIN_ROWS = 136


def _build_triu_tables():
    lens = PER - 1 - np.arange(PER)
    rows = np.repeat(np.arange(PER), lens)
    cols = np.concatenate([np.arange(r + 1, PER) for r in range(PER)])
    rlo = np.empty((WORKERS_PER_GRAPH,), dtype=np.int32)
    nrows = np.empty((WORKERS_PER_GRAPH,), dtype=np.int32)
    idx = np.zeros((WORKERS_PER_GRAPH, OUT_PAD), dtype=np.int32)
    for q in range(WORKERS_PER_GRAPH):
        sl = slice(q * OUT_PER_W, (q + 1) * OUT_PER_W)
        rlo[q] = (rows[sl][0] // 8) * 8
        nrows[q] = -((rlo[q] - (rows[sl][-1] + 1)) // 8) * 8
        assert rows[sl][-1] - rlo[q] < nrows[q] <= WIN_ROWS
        assert rlo[q] + nrows[q] <= PER
        idx[q, :OUT_PER_W] = (rows[sl] - rlo[q]) * PER + cols[sl]
    assert idx.min() >= 0 and idx.max() < WIN_ROWS * PER
    return rlo, nrows, idx


_RLO_NP, _NROWS_NP, _IDX_NP = _build_triu_tables()


def _tc_body(t_ref, idx_ref, x_hbm, wn_hbm, bn_ref, w1_hbm, b1_ref, w2_hbm,
             b2_ref, sims_ref, idxo_hbm,
             xv, wnv, w1v, w2v, sem_idx, sem_wn, sem_w1, sem_w2, *xsems):
    idx_cp = pltpu.make_async_copy(idx_ref, idxo_hbm, sem_idx)
    idx_cp.start()
    w1_cp = pltpu.make_async_copy(w1_hbm, w1v, sem_w1)
    w1_cp.start()
    w2_cp = pltpu.make_async_copy(w2_hbm, w2v, sem_w2)
    w2_cp.start()
    wn_cp = pltpu.make_async_copy(wn_hbm, wnv, sem_wn)
    wn_cp.start()
    def x_copy(g):
        return pltpu.make_async_copy(
            x_hbm.at[pl.ds(g * PER, PER), :],
            xv.at[pl.ds(g * PER, PER), :], xsems[g])

    for g in range(B):
        x_copy(g).start()

    half = TDIM // 2
    freqs_col = jnp.exp(
        lax.broadcasted_iota(jnp.int32, (half, 1), 0).astype(jnp.float32)
        * (-math.log(10000.0) / (half - 1)))
    ang_t = freqs_col * t_ref[...]
    pe_t = jnp.concatenate([jnp.sin(ang_t), jnp.cos(ang_t)], axis=0)
    w1_cp.wait()
    h = lax.dot_general(pe_t, w1v[...], (((0,), (0,)), ((), ())),
                        preferred_element_type=jnp.float32) + b1_ref[...]
    h = h * 0.5 * (1.0 + lax.erf(h / np.sqrt(2.0).astype(np.float32)))
    w2_cp.wait()
    temb = jnp.dot(h, w2v[...], preferred_element_type=jnp.float32) + b2_ref[...]
    c = jnp.sum(temb * temb, axis=1, keepdims=True)

    wn_cp.wait()
    for g in range(B):
        x_copy(g).wait()
        ag = jnp.dot(xv[pl.ds(g * PER, PER), :], wnv[...],
                     preferred_element_type=jnp.float32) + bn_ref[...]
        sims_ref[g] = lax.dot_general(
            ag, ag, (((1,), (1,)), ((), ())),
            preferred_element_type=jnp.float32) + c[g:g + 1, 0:1]
    idx_cp.wait()


def _tc_sims(x, t, W_node, b_node, W1, b1, W2, b2):
    hbm = pl.BlockSpec(memory_space=pltpu.MemorySpace.HBM)
    vmem = pl.BlockSpec(memory_space=pltpu.MemorySpace.VMEM)
    x = pltpu.with_memory_space_constraint(x, pltpu.MemorySpace.HBM)
    W_node = pltpu.with_memory_space_constraint(W_node, pltpu.MemorySpace.HBM)
    W1 = pltpu.with_memory_space_constraint(W1, pltpu.MemorySpace.HBM)
    W2 = pltpu.with_memory_space_constraint(W2, pltpu.MemorySpace.HBM)
    return pl.pallas_call(
        _tc_body,
        in_specs=[vmem, vmem, hbm, hbm, vmem, hbm, vmem, hbm, vmem],
        out_specs=[vmem, hbm],
        out_shape=[jax.ShapeDtypeStruct((B, PER, PER), jnp.float32),
                   pltpu.MemorySpace.HBM((WORKERS_PER_GRAPH * OUT_PAD,),
                                         jnp.int32)],
        scratch_shapes=(
            [pltpu.VMEM((B * PER, FEAT), jnp.float32),
             pltpu.VMEM((FEAT, TDIM), jnp.float32),
             pltpu.VMEM((TDIM, 4 * TDIM), jnp.float32),
             pltpu.VMEM((4 * TDIM, TDIM), jnp.float32)]
            + [pltpu.SemaphoreType.DMA] * 12),
    )(t.reshape(1, B), jnp.asarray(_IDX_NP).reshape(-1), x, W_node,
      b_node.reshape(1, TDIM), W1, b1.reshape(1, 4 * TDIM), W2,
      b2.reshape(1, TDIM))


def _sc_extract_body(sims_hbm, idx_hbm, out_hbm, rows_v, idx_v, out_v,
                     sem_rows, sem_idx):
    wid = lax.axis_index("s") * 2 + lax.axis_index("c")
    g = wid // WORKERS_PER_GRAPH
    q = wid % WORKERS_PER_GRAPH
    idx_cp = pltpu.async_copy(idx_hbm.at[pl.ds(q * OUT_PAD, OUT_PAD)], idx_v,
                              sem_idx)
    for qs in range(WORKERS_PER_GRAPH):
        @pl.when(q == qs)
        def _(qs=qs):
            nr = int(_NROWS_NP[qs])
            pltpu.async_copy(
                sims_hbm.at[g, pl.ds(int(_RLO_NP[qs]), nr), :],
                rows_v.at[pl.ds(0, nr), :], sem_rows).wait()
    idx_cp.wait()

    out_off = g * TRI + q * OUT_PER_W
    HALF = OUT_PAD // 2

    @plsc.parallel_loop(0, HALF, 16, unroll=8)
    def _(i):
        iv = idx_v[pl.ds(i, 16)]
        r = lax.shift_right_logical(iv, 8)
        c = lax.bitwise_and(iv, 255)
        out_v[pl.ds(i, 16)] = plsc.load_gather(rows_v, [r, c])

    out_cp = pltpu.async_copy(out_v.at[pl.ds(0, HALF)],
                              out_hbm.at[pl.ds(out_off, HALF)], sem_rows)

    @plsc.parallel_loop(HALF, OUT_PAD, 16, unroll=8)
    def _(i):
        iv = idx_v[pl.ds(i, 16)]
        r = lax.shift_right_logical(iv, 8)
        c = lax.bitwise_and(iv, 255)
        out_v[pl.ds(i, 16)] = plsc.load_gather(rows_v, [r, c])

    out_cp.wait()
    pltpu.sync_copy(out_v.at[pl.ds(HALF, OUT_PER_W - HALF)],
                    out_hbm.at[pl.ds(out_off + HALF, OUT_PER_W - HALF)])


@functools.lru_cache(maxsize=None)
def _get_sc_extract():
    mesh = plsc.VectorSubcoreMesh(
        core_axis_name="c", subcore_axis_name="s",
        num_cores=2, num_subcores=16)
    return pl.kernel(
        _sc_extract_body,
        out_type=jax.ShapeDtypeStruct((B * TRI,), jnp.float32),
        mesh=mesh,
        compiler_params=pltpu.CompilerParams(
            needs_layout_passes=False,
            disable_bounds_checks=True,
            disable_semaphore_checks=True),
        scratch_types=[
            pltpu.VMEM((WIN_ROWS, PER), jnp.float32),
            pltpu.VMEM((OUT_PAD,), jnp.int32),
            pltpu.VMEM((OUT_PAD,), jnp.float32),
            pltpu.SemaphoreType.DMA,
            pltpu.SemaphoreType.DMA,
        ],
    )


def kernel(x, t, graph_sizes, W_node, b_node, W1, b1, W2, b2):
    sims, idx = _tc_sims(x, t, W_node, b_node, W1, b1, W2, b2)
    sims = pltpu.with_memory_space_constraint(sims, pltpu.MemorySpace.HBM)
    idx = pltpu.with_memory_space_constraint(idx, pltpu.MemorySpace.HBM)
    return _get_sc_extract()(sims, idx)


# speedup vs baseline: 1.0737x; 1.0737x over previous
import functools
import math

import numpy as np
import jax
import jax.numpy as jnp
from jax import lax
from jax.experimental import pallas as pl
from jax.experimental.pallas import tpu as pltpu
from jax.experimental.pallas import tpu_sc as plsc

B = 8
PER = 256
FEAT = 512
TDIM = 256
TRI = PER * (PER - 1) // 2
WORKERS_PER_GRAPH = 4
OUT_PER_W = TRI // WORKERS_PER_GRAPH
OUT_PAD = 8192
WIN_ROWS = 136


def _build_triu_tables():
    lens = PER - 1 - np.arange(PER)
    rows = np.repeat(np.arange(PER), lens)
    cols = np.concatenate([np.arange(r + 1, PER) for r in range(PER)])
    rlo = np.empty((WORKERS_PER_GRAPH,), dtype=np.int32)
    nrows = np.empty((WORKERS_PER_GRAPH,), dtype=np.int32)
    idx = np.zeros((WORKERS_PER_GRAPH, OUT_PAD), dtype=np.int32)
    for q in range(WORKERS_PER_GRAPH):
        sl = slice(q * OUT_PER_W, (q + 1) * OUT_PER_W)
        rlo[q] = (rows[sl][0] // 8) * 8
        nrows[q] = -((rlo[q] - (rows[sl][-1] + 1)) // 8) * 8
        assert rows[sl][-1] - rlo[q] < nrows[q] <= WIN_ROWS
        assert rlo[q] + nrows[q] <= PER
        idx[q, :OUT_PER_W] = (rows[sl] - rlo[q]) * PER + cols[sl]
    assert idx.min() >= 0 and idx.max() < WIN_ROWS * PER
    return rlo, nrows, idx


_RLO_NP, _NROWS_NP, _IDX_NP = _build_triu_tables()


def _tc_body(t_ref, idx_ref, x_ref, wn_ref, bn_ref, w1_ref, b1_ref, w2_ref,
             b2_ref, sims_ref, idxo_hbm, sem_idx):
    idx_cp = pltpu.make_async_copy(idx_ref, idxo_hbm, sem_idx)
    idx_cp.start()

    half = TDIM // 2
    freqs_col = jnp.exp(
        lax.broadcasted_iota(jnp.int32, (half, 1), 0).astype(jnp.float32)
        * (-math.log(10000.0) / (half - 1)))
    ang_t = freqs_col * t_ref[...]
    pe_t = jnp.concatenate([jnp.sin(ang_t), jnp.cos(ang_t)], axis=0)
    h = lax.dot_general(pe_t, w1_ref[...], (((0,), (0,)), ((), ())),
                        preferred_element_type=jnp.float32) + b1_ref[...]
    h = h * 0.5 * (1.0 + lax.erf(h / np.sqrt(2.0).astype(np.float32)))
    temb = jnp.dot(h, w2_ref[...], preferred_element_type=jnp.float32) + b2_ref[...]
    c = jnp.sum(temb * temb, axis=1, keepdims=True)

    for g in range(B):
        ag = jnp.dot(x_ref[pl.ds(g * PER, PER), :], wn_ref[...],
                     preferred_element_type=jnp.float32) + bn_ref[...]
        sims_ref[g] = lax.dot_general(
            ag, ag, (((1,), (1,)), ((), ())),
            preferred_element_type=jnp.float32) + c[g:g + 1, 0:1]
    idx_cp.wait()


def _tc_sims(x, t, W_node, b_node, W1, b1, W2, b2):
    hbm = pl.BlockSpec(memory_space=pltpu.MemorySpace.HBM)
    vmem = pl.BlockSpec(memory_space=pltpu.MemorySpace.VMEM)
    return pl.pallas_call(
        _tc_body,
        in_specs=[vmem, vmem, vmem, vmem, vmem, vmem, vmem, vmem, vmem],
        out_specs=[vmem, hbm],
        out_shape=[jax.ShapeDtypeStruct((B, PER, PER), jnp.float32),
                   pltpu.MemorySpace.HBM((WORKERS_PER_GRAPH * OUT_PAD,),
                                         jnp.int32)],
        scratch_shapes=[
            pltpu.SemaphoreType.DMA,
        ],
        compiler_params=pltpu.CompilerParams(
            disable_bounds_checks=True,
            disable_semaphore_checks=True),
    )(t.reshape(1, B), jnp.asarray(_IDX_NP).reshape(-1), x, W_node,
      b_node.reshape(1, TDIM), W1, b1.reshape(1, 4 * TDIM), W2,
      b2.reshape(1, TDIM))


def _sc_extract_body(sims_hbm, idx_hbm, out_hbm, rows_v, idx_v, out_v,
                     sem_rows, sem_idx):
    wid = lax.axis_index("s") * 2 + lax.axis_index("c")
    g = wid // WORKERS_PER_GRAPH
    q = wid % WORKERS_PER_GRAPH
    idx_cp = pltpu.async_copy(idx_hbm.at[pl.ds(q * OUT_PAD, OUT_PAD)], idx_v,
                              sem_idx)
    for qs in range(WORKERS_PER_GRAPH):
        @pl.when(q == qs)
        def _(qs=qs):
            nr = int(_NROWS_NP[qs])
            pltpu.async_copy(
                sims_hbm.at[g, pl.ds(int(_RLO_NP[qs]), nr), :],
                rows_v.at[pl.ds(0, nr), :], sem_rows).wait()
    idx_cp.wait()

    out_off = g * TRI + q * OUT_PER_W
    HALF = OUT_PAD // 2

    @plsc.parallel_loop(0, HALF, 16, unroll=8)
    def _(i):
        iv = idx_v[pl.ds(i, 16)]
        r = lax.shift_right_logical(iv, 8)
        c = lax.bitwise_and(iv, 255)
        out_v[pl.ds(i, 16)] = plsc.load_gather(rows_v, [r, c])

    out_cp = pltpu.async_copy(out_v.at[pl.ds(0, HALF)],
                              out_hbm.at[pl.ds(out_off, HALF)], sem_rows)

    @plsc.parallel_loop(HALF, OUT_PAD, 16, unroll=8)
    def _(i):
        iv = idx_v[pl.ds(i, 16)]
        r = lax.shift_right_logical(iv, 8)
        c = lax.bitwise_and(iv, 255)
        out_v[pl.ds(i, 16)] = plsc.load_gather(rows_v, [r, c])

    out_cp.wait()
    pltpu.sync_copy(out_v.at[pl.ds(HALF, OUT_PER_W - HALF)],
                    out_hbm.at[pl.ds(out_off + HALF, OUT_PER_W - HALF)])


@functools.lru_cache(maxsize=None)
def _get_sc_extract():
    mesh = plsc.VectorSubcoreMesh(
        core_axis_name="c", subcore_axis_name="s",
        num_cores=2, num_subcores=16)
    return pl.kernel(
        _sc_extract_body,
        out_type=jax.ShapeDtypeStruct((B * TRI,), jnp.float32),
        mesh=mesh,
        compiler_params=pltpu.CompilerParams(
            needs_layout_passes=False,
            disable_bounds_checks=True,
            disable_semaphore_checks=True),
        scratch_types=[
            pltpu.VMEM((WIN_ROWS, PER), jnp.float32),
            pltpu.VMEM((OUT_PAD,), jnp.int32),
            pltpu.VMEM((OUT_PAD,), jnp.float32),
            pltpu.SemaphoreType.DMA,
            pltpu.SemaphoreType.DMA,
        ],
    )


def kernel(x, t, graph_sizes, W_node, b_node, W1, b1, W2, b2):
    sims, idx = _tc_sims(x, t, W_node, b_node, W1, b1, W2, b2)
    sims = pltpu.with_memory_space_constraint(sims, pltpu.MemorySpace.HBM)
    idx = pltpu.with_memory_space_constraint(idx, pltpu.MemorySpace.HBM)
    return _get_sc_extract()(sims, idx)


# speedup vs baseline: 1.0904x; 1.0155x over previous
import functools
import math

import numpy as np
import jax
import jax.numpy as jnp
from jax import lax
from jax.experimental import pallas as pl
from jax.experimental.pallas import tpu as pltpu
from jax.experimental.pallas import tpu_sc as plsc

B = 8
PER = 256
FEAT = 512
TDIM = 256
TRI = PER * (PER - 1) // 2
WORKERS_PER_GRAPH = 4
OUT_PER_W = TRI // WORKERS_PER_GRAPH
OUT_PAD = 8192
WIN_ROWS = 136


def _build_triu_tables():
    lens = PER - 1 - np.arange(PER)
    rows = np.repeat(np.arange(PER), lens)
    cols = np.concatenate([np.arange(r + 1, PER) for r in range(PER)])
    rlo = np.empty((WORKERS_PER_GRAPH,), dtype=np.int32)
    nrows = np.empty((WORKERS_PER_GRAPH,), dtype=np.int32)
    idx = np.zeros((WORKERS_PER_GRAPH, OUT_PAD), dtype=np.int32)
    for q in range(WORKERS_PER_GRAPH):
        sl = slice(q * OUT_PER_W, (q + 1) * OUT_PER_W)
        rlo[q] = (rows[sl][0] // 8) * 8
        nrows[q] = -((rlo[q] - (rows[sl][-1] + 1)) // 8) * 8
        assert rows[sl][-1] - rlo[q] < nrows[q] <= WIN_ROWS
        assert rlo[q] + nrows[q] <= PER
        if q == WORKERS_PER_GRAPH - 1:
            assert cols[sl].min() >= PER // 2
            idx[q, :OUT_PER_W] = ((rows[sl] - rlo[q]) * PER
                                  + (cols[sl] - PER // 2))
        else:
            idx[q, :OUT_PER_W] = (rows[sl] - rlo[q]) * PER + cols[sl]
    assert idx.min() >= 0 and idx.max() < WIN_ROWS * PER
    return rlo, nrows, idx


_RLO_NP, _NROWS_NP, _IDX_NP = _build_triu_tables()


def _tc_body(t_ref, idx_ref, x_ref, wn_ref, bn_ref, w1_ref, b1_ref, w2_ref,
             b2_ref, sims_ref, idxo_hbm, sem_idx):
    idx_cp = pltpu.make_async_copy(idx_ref, idxo_hbm, sem_idx)
    idx_cp.start()

    half = TDIM // 2
    freqs_col = jnp.exp(
        lax.broadcasted_iota(jnp.int32, (half, 1), 0).astype(jnp.float32)
        * (-math.log(10000.0) / (half - 1)))
    ang_t = freqs_col * t_ref[...]
    pe_t = jnp.concatenate([jnp.sin(ang_t), jnp.cos(ang_t)], axis=0)
    h = lax.dot_general(pe_t, w1_ref[...], (((0,), (0,)), ((), ())),
                        preferred_element_type=jnp.float32) + b1_ref[...]
    h = h * 0.5 * (1.0 + lax.erf(h / np.sqrt(2.0).astype(np.float32)))
    temb = jnp.dot(h, w2_ref[...], preferred_element_type=jnp.float32) + b2_ref[...]
    c = jnp.sum(temb * temb, axis=1, keepdims=True)

    for g in range(B):
        ag = jnp.dot(x_ref[pl.ds(g * PER, PER), :], wn_ref[...],
                     preferred_element_type=jnp.float32) + bn_ref[...]
        sims_ref[g] = lax.dot_general(
            ag, ag, (((1,), (1,)), ((), ())),
            preferred_element_type=jnp.float32) + c[g:g + 1, 0:1]
    idx_cp.wait()


def _tc_sims(x, t, W_node, b_node, W1, b1, W2, b2):
    hbm = pl.BlockSpec(memory_space=pltpu.MemorySpace.HBM)
    vmem = pl.BlockSpec(memory_space=pltpu.MemorySpace.VMEM)
    return pl.pallas_call(
        _tc_body,
        in_specs=[vmem, vmem, vmem, vmem, vmem, vmem, vmem, vmem, vmem],
        out_specs=[vmem, hbm],
        out_shape=[jax.ShapeDtypeStruct((B, PER, PER), jnp.float32),
                   pltpu.MemorySpace.HBM((WORKERS_PER_GRAPH * OUT_PAD,),
                                         jnp.int32)],
        scratch_shapes=[
            pltpu.SemaphoreType.DMA,
        ],
        compiler_params=pltpu.CompilerParams(
            disable_bounds_checks=True,
            disable_semaphore_checks=True),
    )(t.reshape(1, B), jnp.asarray(_IDX_NP).reshape(-1), x, W_node,
      b_node.reshape(1, TDIM), W1, b1.reshape(1, 4 * TDIM), W2,
      b2.reshape(1, TDIM))


def _sc_extract_body(sims_hbm, idx_hbm, out_hbm, rows_v, idx_v, out_v,
                     sem_rows, sem_idx):
    wid = lax.axis_index("s") * 2 + lax.axis_index("c")
    g = wid // WORKERS_PER_GRAPH
    q = wid % WORKERS_PER_GRAPH
    idx_cp = pltpu.async_copy(idx_hbm.at[pl.ds(q * OUT_PAD, OUT_PAD)], idx_v,
                              sem_idx)
    for qs in range(WORKERS_PER_GRAPH):
        @pl.when(q == qs)
        def _(qs=qs):
            nr = int(_NROWS_NP[qs])
            if qs == WORKERS_PER_GRAPH - 1:
                pltpu.async_copy(
                    sims_hbm.at[g, pl.ds(int(_RLO_NP[qs]), nr),
                                pl.ds(PER // 2, PER // 2)],
                    rows_v.at[pl.ds(0, nr), pl.ds(0, PER // 2)],
                    sem_rows).wait()
            else:
                pltpu.async_copy(
                    sims_hbm.at[g, pl.ds(int(_RLO_NP[qs]), nr), :],
                    rows_v.at[pl.ds(0, nr), :], sem_rows).wait()
    idx_cp.wait()

    out_off = g * TRI + q * OUT_PER_W
    HALF = OUT_PAD // 2

    @plsc.parallel_loop(0, HALF, 16, unroll=8)
    def _(i):
        iv = idx_v[pl.ds(i, 16)]
        r = lax.shift_right_logical(iv, 8)
        c = lax.bitwise_and(iv, 255)
        out_v[pl.ds(i, 16)] = plsc.load_gather(rows_v, [r, c])

    out_cp = pltpu.async_copy(out_v.at[pl.ds(0, HALF)],
                              out_hbm.at[pl.ds(out_off, HALF)], sem_rows)

    @plsc.parallel_loop(HALF, OUT_PAD, 16, unroll=8)
    def _(i):
        iv = idx_v[pl.ds(i, 16)]
        r = lax.shift_right_logical(iv, 8)
        c = lax.bitwise_and(iv, 255)
        out_v[pl.ds(i, 16)] = plsc.load_gather(rows_v, [r, c])

    out_cp.wait()
    pltpu.sync_copy(out_v.at[pl.ds(HALF, OUT_PER_W - HALF)],
                    out_hbm.at[pl.ds(out_off + HALF, OUT_PER_W - HALF)])


@functools.lru_cache(maxsize=None)
def _get_sc_extract():
    mesh = plsc.VectorSubcoreMesh(
        core_axis_name="c", subcore_axis_name="s",
        num_cores=2, num_subcores=16)
    return pl.kernel(
        _sc_extract_body,
        out_type=jax.ShapeDtypeStruct((B * TRI,), jnp.float32),
        mesh=mesh,
        compiler_params=pltpu.CompilerParams(
            needs_layout_passes=False,
            disable_bounds_checks=True,
            disable_semaphore_checks=True),
        scratch_types=[
            pltpu.VMEM((WIN_ROWS, PER), jnp.float32),
            pltpu.VMEM((OUT_PAD,), jnp.int32),
            pltpu.VMEM((OUT_PAD,), jnp.float32),
            pltpu.SemaphoreType.DMA,
            pltpu.SemaphoreType.DMA,
        ],
    )


def kernel(x, t, graph_sizes, W_node, b_node, W1, b1, W2, b2):
    sims, idx = _tc_sims(x, t, W_node, b_node, W1, b1, W2, b2)
    sims = pltpu.with_memory_space_constraint(sims, pltpu.MemorySpace.HBM)
    idx = pltpu.with_memory_space_constraint(idx, pltpu.MemorySpace.HBM)
    return _get_sc_extract()(sims, idx)


# speedup vs baseline: 1.0960x; 1.0051x over previous
import functools
import math

import numpy as np
import jax
import jax.numpy as jnp
from jax import lax
from jax.experimental import pallas as pl
from jax.experimental.pallas import tpu as pltpu
from jax.experimental.pallas import tpu_sc as plsc

B = 8
PER = 256
FEAT = 512
TDIM = 256
TRI = PER * (PER - 1) // 2
WORKERS_PER_GRAPH = 4
OUT_PER_W = TRI // WORKERS_PER_GRAPH
OUT_PAD = 8192
WIN_ROWS = 136


def _build_triu_tables():
    lens = PER - 1 - np.arange(PER)
    rows = np.repeat(np.arange(PER), lens)
    cols = np.concatenate([np.arange(r + 1, PER) for r in range(PER)])
    rlo = np.empty((WORKERS_PER_GRAPH,), dtype=np.int32)
    nrows = np.empty((WORKERS_PER_GRAPH,), dtype=np.int32)
    nr1 = np.empty((WORKERS_PER_GRAPH,), dtype=np.int32)
    idx = np.zeros((WORKERS_PER_GRAPH, OUT_PAD), dtype=np.int32)
    for q in range(WORKERS_PER_GRAPH):
        sl = slice(q * OUT_PER_W, (q + 1) * OUT_PER_W)
        rlo[q] = (rows[sl][0] // 8) * 8
        nrows[q] = -((rlo[q] - (rows[sl][-1] + 1)) // 8) * 8
        nr1[q] = -((rlo[q] - (rows[q * OUT_PER_W + OUT_PAD // 2 - 1] + 1))
                   // 8) * 8
        assert rows[sl][-1] - rlo[q] < nrows[q] <= WIN_ROWS
        assert 0 < nr1[q] <= nrows[q]
        assert rlo[q] + nrows[q] <= PER
        if q == WORKERS_PER_GRAPH - 1:
            assert cols[sl].min() >= PER // 2
            idx[q, :OUT_PER_W] = ((rows[sl] - rlo[q]) * PER
                                  + (cols[sl] - PER // 2))
        else:
            idx[q, :OUT_PER_W] = (rows[sl] - rlo[q]) * PER + cols[sl]
    assert idx.min() >= 0 and idx.max() < WIN_ROWS * PER
    return rlo, nrows, nr1, idx


_RLO_NP, _NROWS_NP, _NR1_NP, _IDX_NP = _build_triu_tables()


def _tc_body(t_ref, idx_ref, x_ref, wn_ref, bn_ref, w1_ref, b1_ref, w2_ref,
             b2_ref, sims_ref, idxo_hbm, sem_idx):
    idx_cp = pltpu.make_async_copy(idx_ref, idxo_hbm, sem_idx)
    idx_cp.start()

    half = TDIM // 2
    freqs_col = jnp.exp(
        lax.broadcasted_iota(jnp.int32, (half, 1), 0).astype(jnp.float32)
        * (-math.log(10000.0) / (half - 1)))
    ang_t = freqs_col * t_ref[...]
    pe_t = jnp.concatenate([jnp.sin(ang_t), jnp.cos(ang_t)], axis=0)
    h = lax.dot_general(pe_t, w1_ref[...], (((0,), (0,)), ((), ())),
                        preferred_element_type=jnp.float32) + b1_ref[...]
    h = h * 0.5 * (1.0 + lax.erf(h / np.sqrt(2.0).astype(np.float32)))
    temb = jnp.dot(h, w2_ref[...], preferred_element_type=jnp.float32) + b2_ref[...]
    c = jnp.sum(temb * temb, axis=1, keepdims=True)

    for g in range(B):
        ag = jnp.dot(x_ref[pl.ds(g * PER, PER), :], wn_ref[...],
                     preferred_element_type=jnp.float32) + bn_ref[...]
        sims_ref[g] = lax.dot_general(
            ag, ag, (((1,), (1,)), ((), ())),
            preferred_element_type=jnp.float32) + c[g:g + 1, 0:1]
    idx_cp.wait()


def _tc_sims(x, t, W_node, b_node, W1, b1, W2, b2):
    hbm = pl.BlockSpec(memory_space=pltpu.MemorySpace.HBM)
    vmem = pl.BlockSpec(memory_space=pltpu.MemorySpace.VMEM)
    return pl.pallas_call(
        _tc_body,
        in_specs=[vmem, vmem, vmem, vmem, vmem, vmem, vmem, vmem, vmem],
        out_specs=[vmem, hbm],
        out_shape=[jax.ShapeDtypeStruct((B, PER, PER), jnp.float32),
                   pltpu.MemorySpace.HBM((WORKERS_PER_GRAPH * OUT_PAD,),
                                         jnp.int32)],
        scratch_shapes=[
            pltpu.SemaphoreType.DMA,
        ],
        compiler_params=pltpu.CompilerParams(
            disable_bounds_checks=True,
            disable_semaphore_checks=True),
    )(t.reshape(1, B), jnp.asarray(_IDX_NP).reshape(-1), x, W_node,
      b_node.reshape(1, TDIM), W1, b1.reshape(1, 4 * TDIM), W2,
      b2.reshape(1, TDIM))


def _sc_extract_body(sims_hbm, idx_hbm, out_hbm, rows_v, idx_v, out_v,
                     sem_rows, sem_idx, sem_r2):
    wid = lax.axis_index("s") * 2 + lax.axis_index("c")
    g = wid // WORKERS_PER_GRAPH
    q = wid % WORKERS_PER_GRAPH
    idx_cp = pltpu.async_copy(idx_hbm.at[pl.ds(q * OUT_PAD, OUT_PAD)], idx_v,
                              sem_idx)
    def row_copy(qs, r0, nr, sem):
        if qs == WORKERS_PER_GRAPH - 1:
            return pltpu.make_async_copy(
                sims_hbm.at[g, pl.ds(int(_RLO_NP[qs]) + r0, nr),
                            pl.ds(PER // 2, PER // 2)],
                rows_v.at[pl.ds(r0, nr), pl.ds(0, PER // 2)], sem)
        return pltpu.make_async_copy(
            sims_hbm.at[g, pl.ds(int(_RLO_NP[qs]) + r0, nr), :],
            rows_v.at[pl.ds(r0, nr), :], sem)

    def stage2(qs):
        n1 = int(_NR1_NP[qs])
        return row_copy(qs, n1, int(_NROWS_NP[qs]) - n1, sem_r2)

    for qs in range(WORKERS_PER_GRAPH):
        @pl.when(q == qs)
        def _(qs=qs):
            stage2(qs).start()
            cp1 = row_copy(qs, 0, int(_NR1_NP[qs]), sem_rows)
            cp1.start()
            cp1.wait()
    idx_cp.wait()

    out_off = g * TRI + q * OUT_PER_W
    HALF = OUT_PAD // 2

    @plsc.parallel_loop(0, HALF, 16, unroll=8)
    def _(i):
        iv = idx_v[pl.ds(i, 16)]
        r = lax.shift_right_logical(iv, 8)
        c = lax.bitwise_and(iv, 255)
        out_v[pl.ds(i, 16)] = plsc.load_gather(rows_v, [r, c])

    for qs in range(WORKERS_PER_GRAPH):
        @pl.when(q == qs)
        def _(qs=qs):
            stage2(qs).wait()

    out_cp = pltpu.async_copy(out_v.at[pl.ds(0, HALF)],
                              out_hbm.at[pl.ds(out_off, HALF)], sem_rows)

    @plsc.parallel_loop(HALF, OUT_PAD, 16, unroll=8)
    def _(i):
        iv = idx_v[pl.ds(i, 16)]
        r = lax.shift_right_logical(iv, 8)
        c = lax.bitwise_and(iv, 255)
        out_v[pl.ds(i, 16)] = plsc.load_gather(rows_v, [r, c])

    out_cp.wait()
    pltpu.sync_copy(out_v.at[pl.ds(HALF, OUT_PER_W - HALF)],
                    out_hbm.at[pl.ds(out_off + HALF, OUT_PER_W - HALF)])


@functools.lru_cache(maxsize=None)
def _get_sc_extract():
    mesh = plsc.VectorSubcoreMesh(
        core_axis_name="c", subcore_axis_name="s",
        num_cores=2, num_subcores=16)
    return pl.kernel(
        _sc_extract_body,
        out_type=jax.ShapeDtypeStruct((B * TRI,), jnp.float32),
        mesh=mesh,
        compiler_params=pltpu.CompilerParams(
            needs_layout_passes=False,
            disable_bounds_checks=True,
            disable_semaphore_checks=True),
        scratch_types=[
            pltpu.VMEM((WIN_ROWS, PER), jnp.float32),
            pltpu.VMEM((OUT_PAD,), jnp.int32),
            pltpu.VMEM((OUT_PAD,), jnp.float32),
            pltpu.SemaphoreType.DMA,
            pltpu.SemaphoreType.DMA,
            pltpu.SemaphoreType.DMA,
        ],
    )


def kernel(x, t, graph_sizes, W_node, b_node, W1, b1, W2, b2):
    sims, idx = _tc_sims(x, t, W_node, b_node, W1, b1, W2, b2)
    sims = pltpu.with_memory_space_constraint(sims, pltpu.MemorySpace.HBM)
    idx = pltpu.with_memory_space_constraint(idx, pltpu.MemorySpace.HBM)
    return _get_sc_extract()(sims, idx)
